# baseline (speedup 1.0000x reference)
import functools

import jax
import jax.numpy as jnp
from jax import lax
from jax.experimental import pallas as pl
from jax.experimental.pallas import tpu as pltpu
from jax.experimental.pallas import tpu_sc as plsc

BATCH = 4
SEQ_LEN = 8192
D_MODEL = 768


BS = 1024


def _tc_body(x_ref, pos_ref, out_ref):
    out_ref[...] = x_ref[...] + pos_ref[...][None]


def _tc_kernel(x, pos_table):
    grid = (SEQ_LEN // BS,)
    return pl.pallas_call(
        _tc_body,
        grid=grid,
        in_specs=[
            pl.BlockSpec((BATCH, BS, D_MODEL), lambda s: (0, s, 0)),
            pl.BlockSpec((BS, D_MODEL), lambda s: (s, 0)),
        ],
        out_specs=pl.BlockSpec((BATCH, BS, D_MODEL), lambda s: (0, s, 0)),
        out_shape=jax.ShapeDtypeStruct((BATCH, SEQ_LEN, D_MODEL), jnp.float32),
    )(x, pos_table)


_SC_NC = 2
_SC_NS = 16
_SC_NW = _SC_NC * _SC_NS
_SC_RPW = SEQ_LEN // _SC_NW
_SC_CH = 64
_SC_CHW = _SC_CH * D_MODEL
_SC_NCHUNK = _SC_RPW // _SC_CH


def _sc_body(x_hbm, pos_hbm, out_hbm, x_v, pos_v):
    wid = lax.axis_index("s") * _SC_NC + lax.axis_index("c")
    base = wid * _SC_RPW * D_MODEL
    for c in range(_SC_NCHUNK):
        off = base + c * _SC_CHW
        pltpu.sync_copy(pos_hbm.at[pl.ds(off, _SC_CHW)], pos_v)
        for b in range(BATCH):
            xoff = b * SEQ_LEN * D_MODEL + off
            pltpu.sync_copy(x_hbm.at[pl.ds(xoff, _SC_CHW)], x_v)

            @plsc.parallel_loop(0, _SC_CHW, step=16, unroll=8)
            def _add(i):
                x_v[pl.ds(i, 16)] = x_v[pl.ds(i, 16)] + pos_v[pl.ds(i, 16)]

            pltpu.sync_copy(x_v, out_hbm.at[pl.ds(xoff, _SC_CHW)])


def _sc_kernel(x1, pos1):
    mesh = plsc.VectorSubcoreMesh(core_axis_name="c", subcore_axis_name="s")
    run = pl.kernel(
        _sc_body,
        out_type=jax.ShapeDtypeStruct((BATCH * SEQ_LEN * D_MODEL,), jnp.float32),
        mesh=mesh,
        scratch_types=[
            pltpu.VMEM((_SC_CHW,), jnp.float32),
            pltpu.VMEM((_SC_CHW,), jnp.float32),
        ],
    )
    return run(x1, pos1)


def kernel(x, pos_table):
    x1 = x.reshape(BATCH * SEQ_LEN * D_MODEL)
    pos1 = pos_table.reshape(SEQ_LEN * D_MODEL)
    out = _sc_kernel(x1, pos1)
    return out.reshape(BATCH, SEQ_LEN, D_MODEL)


# speedup vs baseline: 1.0970x; 1.0970x over previous
import functools

import jax
import jax.numpy as jnp
from jax import lax
from jax.experimental import pallas as pl
from jax.experimental.pallas import tpu as pltpu
from jax.experimental.pallas import tpu_sc as plsc

BATCH = 4
SEQ_LEN = 8192
D_MODEL = 768


BS = 1024


def _tc_body(x_ref, pos_ref, out_ref):
    out_ref[...] = x_ref[...] + pos_ref[...][None]


def _tc_kernel(x, pos_table):
    grid = (SEQ_LEN // BS,)
    return pl.pallas_call(
        _tc_body,
        grid=grid,
        in_specs=[
            pl.BlockSpec((BATCH, BS, D_MODEL), lambda s: (0, s, 0)),
            pl.BlockSpec((BS, D_MODEL), lambda s: (s, 0)),
        ],
        out_specs=pl.BlockSpec((BATCH, BS, D_MODEL), lambda s: (0, s, 0)),
        out_shape=jax.ShapeDtypeStruct((BATCH, SEQ_LEN, D_MODEL), jnp.float32),
    )(x, pos_table)


_SC_NC = 2
_SC_NS = 16
_SC_NW = _SC_NC * _SC_NS
_SC_RPW = SEQ_LEN // _SC_NW
_SC_CH = 32
_SC_CHW = _SC_CH * D_MODEL
_SC_NCHUNK = _SC_RPW // _SC_CH


def _sc_body(x_hbm, pos_hbm, out_hbm, x_v0, x_v1, pos_v,
             li0, li1, lo0, lo1):
    wid = lax.axis_index("s") * _SC_NC + lax.axis_index("c")
    base = wid * _SC_RPW * D_MODEL
    bufs = (x_v0, x_v1)
    lsems = (li0, li1)
    ssems = (lo0, lo1)

    steps = [(c, b) for c in range(_SC_NCHUNK) for b in range(BATCH)]

    def xoff(step):
        c, b = steps[step]
        return b * SEQ_LEN * D_MODEL + base + c * _SC_CHW

    nsteps = len(steps)
    loads = [None] * nsteps
    stores = [None] * nsteps

    def start_load(t):
        buf = bufs[t % 2]
        loads[t] = pltpu.async_copy(
            x_hbm.at[pl.ds(xoff(t), _SC_CHW)], buf, lsems[t % 2])

    start_load(0)
    for t in range(nsteps):
        cur = bufs[t % 2]
        if t + 1 < nsteps:
            if t - 1 >= 0:
                stores[t - 1].wait()
            start_load(t + 1)
        if steps[t][1] == 0:
            pltpu.sync_copy(
                pos_hbm.at[pl.ds(base + steps[t][0] * _SC_CHW, _SC_CHW)], pos_v)
        loads[t].wait()

        @plsc.parallel_loop(0, _SC_CHW, step=16, unroll=8)
        def _add(i):
            cur[pl.ds(i, 16)] = cur[pl.ds(i, 16)] + pos_v[pl.ds(i, 16)]

        stores[t] = pltpu.async_copy(
            cur, out_hbm.at[pl.ds(xoff(t), _SC_CHW)], ssems[t % 2])
    stores[nsteps - 2].wait()
    stores[nsteps - 1].wait()


def _sc_kernel(x1, pos1):
    mesh = plsc.VectorSubcoreMesh(core_axis_name="c", subcore_axis_name="s")
    run = pl.kernel(
        _sc_body,
        out_type=jax.ShapeDtypeStruct((BATCH * SEQ_LEN * D_MODEL,), jnp.float32),
        mesh=mesh,
        scratch_types=[
            pltpu.VMEM((_SC_CHW,), jnp.float32),
            pltpu.VMEM((_SC_CHW,), jnp.float32),
            pltpu.VMEM((_SC_CHW,), jnp.float32),
            pltpu.SemaphoreType.DMA,
            pltpu.SemaphoreType.DMA,
            pltpu.SemaphoreType.DMA,
            pltpu.SemaphoreType.DMA,
        ],
    )
    return run(x1, pos1)


def kernel(x, pos_table):
    x1 = x.reshape(BATCH * SEQ_LEN * D_MODEL)
    pos1 = pos_table.reshape(SEQ_LEN * D_MODEL)
    out = _sc_kernel(x1, pos1)
    return out.reshape(BATCH, SEQ_LEN, D_MODEL)


# speedup vs baseline: 1.2818x; 1.1685x over previous
import jax
import jax.numpy as jnp
from jax import lax
from jax.experimental import pallas as pl
from jax.experimental.pallas import tpu as pltpu
from jax.experimental.pallas import tpu_sc as plsc

BATCH = 4
SEQ_LEN = 8192
D_MODEL = 768

_S0 = 4096


BS = 1024


def _tc_main_body(x_ref, pos_ref, out_ref):
    out_ref[...] = x_ref[...] + pos_ref[...][None]


def _tc_main(x, pos_table):
    grid = (SEQ_LEN // BS,)
    return pl.pallas_call(
        _tc_main_body,
        grid=grid,
        in_specs=[
            pl.BlockSpec((BATCH - 1, BS, D_MODEL), lambda s: (0, s, 0)),
            pl.BlockSpec((BS, D_MODEL), lambda s: (s, 0)),
        ],
        out_specs=pl.BlockSpec((BATCH - 1, BS, D_MODEL), lambda s: (0, s, 0)),
        out_shape=jax.ShapeDtypeStruct((BATCH - 1, SEQ_LEN, D_MODEL), jnp.float32),
    )(x, pos_table)


def _tc_head_body(x_ref, pos_ref, out_ref):
    out_ref[...] = x_ref[0] + pos_ref[...]


def _tc_head(x, pos_table):
    grid = (_S0 // BS,)
    return pl.pallas_call(
        _tc_head_body,
        grid=grid,
        in_specs=[
            pl.BlockSpec((1, BS, D_MODEL), lambda s: (BATCH - 1, s, 0)),
            pl.BlockSpec((BS, D_MODEL), lambda s: (s, 0)),
        ],
        out_specs=pl.BlockSpec((BS, D_MODEL), lambda s: (s, 0)),
        out_shape=jax.ShapeDtypeStruct((_S0, D_MODEL), jnp.float32),
    )(x, pos_table)


_SC_NC = 2
_SC_NS = 16
_SC_NW = _SC_NC * _SC_NS
_SC_ROWS = SEQ_LEN - _S0
_SC_RPW = _SC_ROWS // _SC_NW
_SC_CH = 32
_SC_CHW = _SC_CH * D_MODEL
_SC_NCHUNK = _SC_RPW // _SC_CH


def _sc_body(x_hbm, pos_hbm, out_hbm, x_v0, x_v1, pos_v,
             li0, li1, lo0, lo1):
    wid = lax.axis_index("s") * _SC_NC + lax.axis_index("c")
    seq_base = (_S0 + wid * _SC_RPW) * D_MODEL
    x_base = (BATCH - 1) * SEQ_LEN * D_MODEL + seq_base
    out_base = wid * _SC_RPW * D_MODEL
    bufs = (x_v0, x_v1)
    lsems = (li0, li1)
    ssems = (lo0, lo1)

    nsteps = _SC_NCHUNK
    loads = [None] * nsteps
    stores = [None] * nsteps

    def start_load(t):
        loads[t] = pltpu.async_copy(
            x_hbm.at[pl.ds(x_base + t * _SC_CHW, _SC_CHW)],
            bufs[t % 2], lsems[t % 2])

    start_load(0)
    for t in range(nsteps):
        cur = bufs[t % 2]
        if t + 1 < nsteps:
            if t - 1 >= 0:
                stores[t - 1].wait()
            start_load(t + 1)
        pltpu.sync_copy(pos_hbm.at[pl.ds(seq_base + t * _SC_CHW, _SC_CHW)],
                        pos_v)
        loads[t].wait()

        @plsc.parallel_loop(0, _SC_CHW, step=16, unroll=8)
        def _add(i):
            cur[pl.ds(i, 16)] = cur[pl.ds(i, 16)] + pos_v[pl.ds(i, 16)]

        stores[t] = pltpu.async_copy(
            cur, out_hbm.at[pl.ds(out_base + t * _SC_CHW, _SC_CHW)],
            ssems[t % 2])
    if nsteps >= 2:
        stores[nsteps - 2].wait()
    stores[nsteps - 1].wait()


def _sc_tail(x1, pos1):
    mesh = plsc.VectorSubcoreMesh(core_axis_name="c", subcore_axis_name="s")
    run = pl.kernel(
        _sc_body,
        out_type=jax.ShapeDtypeStruct((_SC_ROWS * D_MODEL,), jnp.float32),
        mesh=mesh,
        scratch_types=[
            pltpu.VMEM((_SC_CHW,), jnp.float32),
            pltpu.VMEM((_SC_CHW,), jnp.float32),
            pltpu.VMEM((_SC_CHW,), jnp.float32),
            pltpu.SemaphoreType.DMA,
            pltpu.SemaphoreType.DMA,
            pltpu.SemaphoreType.DMA,
            pltpu.SemaphoreType.DMA,
        ],
    )
    return run(x1, pos1)


def kernel(x, pos_table):
    x1 = x.reshape(BATCH * SEQ_LEN * D_MODEL)
    pos1 = pos_table.reshape(SEQ_LEN * D_MODEL)
    a = _tc_main(x, pos_table)
    b = _tc_head(x, pos_table)
    c = _sc_tail(x1, pos1)
    out2 = jnp.concatenate(
        [
            a.reshape((BATCH - 1) * SEQ_LEN, D_MODEL),
            b,
            c.reshape(_SC_ROWS, D_MODEL),
        ],
        axis=0,
    )
    return out2.reshape(BATCH, SEQ_LEN, D_MODEL)


# speedup vs baseline: 3.9288x; 3.0650x over previous
import jax
import jax.numpy as jnp
from jax import lax
from jax.experimental import pallas as pl
from jax.experimental.pallas import tpu as pltpu
from jax.experimental.pallas import tpu_sc as plsc

BATCH = 4
SEQ_LEN = 8192
D_MODEL = 768

BS = 1024
_SC_ROWS = 2048
_S0 = SEQ_LEN - _SC_ROWS


_SC_NC = 2
_SC_NS = 16
_SC_NW = _SC_NC * _SC_NS
_SC_RPW = _SC_ROWS // _SC_NW
_SC_CH = 32
_SC_NCHUNK = _SC_RPW // _SC_CH


def _sc_body(x_hbm, pos_hbm, out_hbm, x_v0, x_v1, pos_v,
             li0, li1, lo0, lo1):
    wid = lax.axis_index("s") * _SC_NC + lax.axis_index("c")
    seq0 = _S0 + wid * _SC_RPW
    out0 = wid * _SC_RPW
    bufs = (x_v0, x_v1)
    lsems = (li0, li1)
    ssems = (lo0, lo1)

    nsteps = _SC_NCHUNK
    loads = [None] * nsteps
    stores = [None] * nsteps

    def start_load(t):
        loads[t] = pltpu.async_copy(
            x_hbm.at[BATCH - 1, pl.ds(seq0 + t * _SC_CH, _SC_CH)],
            bufs[t % 2], lsems[t % 2])

    start_load(0)
    for t in range(nsteps):
        cur = bufs[t % 2]
        if t + 1 < nsteps:
            if t - 1 >= 0:
                stores[t - 1].wait()
            start_load(t + 1)
        pltpu.sync_copy(pos_hbm.at[pl.ds(seq0 + t * _SC_CH, _SC_CH)], pos_v)
        loads[t].wait()

        @plsc.parallel_loop(0, _SC_CH, step=1, unroll=2)
        def _add(r):
            for c in range(D_MODEL // 16):
                sl = pl.ds(c * 16, 16)
                cur[r, sl] = cur[r, sl] + pos_v[r, sl]

        stores[t] = pltpu.async_copy(
            cur, out_hbm.at[pl.ds(out0 + t * _SC_CH, _SC_CH)], ssems[t % 2])
    if nsteps >= 2:
        stores[nsteps - 2].wait()
    stores[nsteps - 1].wait()


def _sc_tail(x, pos_table):
    mesh = plsc.VectorSubcoreMesh(core_axis_name="c", subcore_axis_name="s")
    run = pl.kernel(
        _sc_body,
        out_type=jax.ShapeDtypeStruct((_SC_ROWS, D_MODEL), jnp.float32),
        mesh=mesh,
        scratch_types=[
            pltpu.VMEM((_SC_CH, D_MODEL), jnp.float32),
            pltpu.VMEM((_SC_CH, D_MODEL), jnp.float32),
            pltpu.VMEM((_SC_CH, D_MODEL), jnp.float32),
            pltpu.SemaphoreType.DMA,
            pltpu.SemaphoreType.DMA,
            pltpu.SemaphoreType.DMA,
            pltpu.SemaphoreType.DMA,
        ],
    )
    return run(x, pos_table)


_N_SEQ = SEQ_LEN // BS
_N_HEAD = _S0 // BS
_N_STEPS = (BATCH - 1) * _N_SEQ + _N_HEAD
_FULL = _N_HEAD * BATCH


def _tc_pb(i):
    p = jnp.where(i < _FULL, i // BATCH, _N_HEAD + (i - _FULL) // (BATCH - 1))
    b = jnp.where(i < _FULL, i % BATCH, (i - _FULL) % (BATCH - 1))
    return p, b


def _tc_main_body(x_ref, pos_ref, out_ref):
    out_ref[...] = x_ref[...] + pos_ref[...][None]


def _tc_main(x, pos_table):
    return pl.pallas_call(
        _tc_main_body,
        grid=(_N_STEPS,),
        in_specs=[
            pl.BlockSpec((1, BS, D_MODEL),
                         lambda i: (_tc_pb(i)[1], _tc_pb(i)[0], 0)),
            pl.BlockSpec((BS, D_MODEL), lambda i: (_tc_pb(i)[0], 0)),
        ],
        out_specs=pl.BlockSpec((1, BS, D_MODEL),
                               lambda i: (_tc_pb(i)[1], _tc_pb(i)[0], 0)),
        out_shape=jax.ShapeDtypeStruct((BATCH, SEQ_LEN, D_MODEL), jnp.float32),
    )(x, pos_table)


def _patch_body(main_ref, sc_ref, out_ref):
    out_ref[...] = sc_ref[...][None]


def _patch(main, sc_out):
    return pl.pallas_call(
        _patch_body,
        grid=(_SC_ROWS // BS,),
        in_specs=[
            pl.BlockSpec(memory_space=pl.ANY),
            pl.BlockSpec((BS, D_MODEL), lambda s: (s, 0)),
        ],
        out_specs=pl.BlockSpec((1, BS, D_MODEL),
                               lambda s: (BATCH - 1, _N_HEAD + s, 0)),
        out_shape=jax.ShapeDtypeStruct((BATCH, SEQ_LEN, D_MODEL), jnp.float32),
        input_output_aliases={0: 0},
    )(main, sc_out)


def kernel(x, pos_table):
    sc_out = _sc_tail(x, pos_table)
    main = _tc_main(x, pos_table)
    return _patch(main, sc_out)


# speedup vs baseline: 4.0821x; 1.0390x over previous
import jax
import jax.numpy as jnp
from jax import lax
from jax.experimental import pallas as pl
from jax.experimental.pallas import tpu as pltpu
from jax.experimental.pallas import tpu_sc as plsc

BATCH = 4
SEQ_LEN = 8192
D_MODEL = 768

BS = 2048
_SC_ROWS = 2048
_S0 = SEQ_LEN - _SC_ROWS


_SC_NC = 2
_SC_NS = 16
_SC_NW = _SC_NC * _SC_NS
_SC_RPW = _SC_ROWS // _SC_NW
_SC_CH = 32
_SC_NCHUNK = _SC_RPW // _SC_CH


def _sc_body(x_hbm, pos_hbm, out_hbm, x_v0, x_v1, pos_v,
             li0, li1, lo0, lo1):
    wid = lax.axis_index("s") * _SC_NC + lax.axis_index("c")
    seq0 = _S0 + wid * _SC_RPW
    out0 = wid * _SC_RPW
    bufs = (x_v0, x_v1)
    lsems = (li0, li1)
    ssems = (lo0, lo1)

    nsteps = _SC_NCHUNK
    loads = [None] * nsteps
    stores = [None] * nsteps

    def start_load(t):
        loads[t] = pltpu.async_copy(
            x_hbm.at[BATCH - 1, pl.ds(seq0 + t * _SC_CH, _SC_CH)],
            bufs[t % 2], lsems[t % 2])

    start_load(0)
    for t in range(nsteps):
        cur = bufs[t % 2]
        if t + 1 < nsteps:
            if t - 1 >= 0:
                stores[t - 1].wait()
            start_load(t + 1)
        pltpu.sync_copy(pos_hbm.at[pl.ds(seq0 + t * _SC_CH, _SC_CH)], pos_v)
        loads[t].wait()

        @plsc.parallel_loop(0, _SC_CH, step=1, unroll=2)
        def _add(r):
            for c in range(D_MODEL // 16):
                sl = pl.ds(c * 16, 16)
                cur[r, sl] = cur[r, sl] + pos_v[r, sl]

        stores[t] = pltpu.async_copy(
            cur, out_hbm.at[pl.ds(out0 + t * _SC_CH, _SC_CH)], ssems[t % 2])
    if nsteps >= 2:
        stores[nsteps - 2].wait()
    stores[nsteps - 1].wait()


def _sc_tail(x, pos_table):
    mesh = plsc.VectorSubcoreMesh(core_axis_name="c", subcore_axis_name="s")
    run = pl.kernel(
        _sc_body,
        out_type=jax.ShapeDtypeStruct((_SC_ROWS, D_MODEL), jnp.float32),
        mesh=mesh,
        scratch_types=[
            pltpu.VMEM((_SC_CH, D_MODEL), jnp.float32),
            pltpu.VMEM((_SC_CH, D_MODEL), jnp.float32),
            pltpu.VMEM((_SC_CH, D_MODEL), jnp.float32),
            pltpu.SemaphoreType.DMA,
            pltpu.SemaphoreType.DMA,
            pltpu.SemaphoreType.DMA,
            pltpu.SemaphoreType.DMA,
        ],
    )
    return run(x, pos_table)


_N_SEQ = SEQ_LEN // BS
_N_HEAD = _S0 // BS
_N_STEPS = (BATCH - 1) * _N_SEQ + _N_HEAD
_FULL = _N_HEAD * BATCH


def _tc_pb(i):
    p = jnp.where(i < _FULL, i // BATCH, _N_HEAD + (i - _FULL) // (BATCH - 1))
    b = jnp.where(i < _FULL, i % BATCH, (i - _FULL) % (BATCH - 1))
    return p, b


def _tc_main_body(x_ref, pos_ref, out_ref):
    out_ref[...] = x_ref[...] + pos_ref[...][None]


def _tc_main(x, pos_table):
    return pl.pallas_call(
        _tc_main_body,
        grid=(_N_STEPS,),
        in_specs=[
            pl.BlockSpec((1, BS, D_MODEL),
                         lambda i: (_tc_pb(i)[1], _tc_pb(i)[0], 0)),
            pl.BlockSpec((BS, D_MODEL), lambda i: (_tc_pb(i)[0], 0)),
        ],
        out_specs=pl.BlockSpec((1, BS, D_MODEL),
                               lambda i: (_tc_pb(i)[1], _tc_pb(i)[0], 0)),
        out_shape=jax.ShapeDtypeStruct((BATCH, SEQ_LEN, D_MODEL), jnp.float32),
    )(x, pos_table)


def _patch_body(main_ref, sc_ref, out_ref):
    out_ref[...] = sc_ref[...][None]


def _patch(main, sc_out):
    return pl.pallas_call(
        _patch_body,
        grid=(_SC_ROWS // BS,),
        in_specs=[
            pl.BlockSpec(memory_space=pl.ANY),
            pl.BlockSpec((BS, D_MODEL), lambda s: (s, 0)),
        ],
        out_specs=pl.BlockSpec((1, BS, D_MODEL),
                               lambda s: (BATCH - 1, _N_HEAD + s, 0)),
        out_shape=jax.ShapeDtypeStruct((BATCH, SEQ_LEN, D_MODEL), jnp.float32),
        input_output_aliases={0: 0},
    )(main, sc_out)


def kernel(x, pos_table):
    sc_out = _sc_tail(x, pos_table)
    main = _tc_main(x, pos_table)
    return _patch(main, sc_out)
